# baseline (speedup 1.0000x reference)
import functools

import jax
import jax.numpy as jnp
from jax import lax
from jax.experimental import pallas as pl
from jax.experimental.pallas import tpu as pltpu
from jax.experimental.pallas import tpu_sc as plsc

N = 10000
NPAD = 10240
E0 = 330000
NW = 32
EPT = 10368
EPAD = NW * EPT
ROWS_PER_TILE = NPAD // 16
PACK = 16384


def _make_sc_gat(H, WP, AW, B, NB, dual, nb_split=None):
    assert NB % 3 == 0
    n_pass = 2 if dual else 1
    nb_max = NB if nb_split is None else max(nb_split)
    mesh = plsc.VectorSubcoreMesh(core_axis_name="c", subcore_axis_name="s")
    scratch = [pltpu.VMEM((nb_max, B), jnp.int32)]
    scratch += [pltpu.VMEM((1, B), jnp.int32)] * 6
    scratch += [pltpu.VMEM((B, WP), jnp.float32)] * 3
    scratch += [pltpu.VMEM((B, AW), jnp.float32)] * 3
    if H == 1:
        scratch += [pltpu.VMEM((NPAD,), jnp.float32)] * 2
        scratch += [pltpu.VMEM((B * 16,), jnp.float32)] * 3
    else:
        scratch += [pltpu.VMEM((B, 16), jnp.float32)] * 6
    scratch += [pltpu.VMEM_SHARED((NPAD, AW), jnp.float32)]
    scratch += [pltpu.SemaphoreType.DMA] * (6 + (6 if H > 1 else 0))

    out_one = jax.ShapeDtypeStruct((2, NPAD, AW), jnp.float32)

    @functools.partial(
        pl.kernel,
        out_type=tuple([out_one] * n_pass) if dual else out_one,
        mesh=mesh,
        scratch_types=scratch,
        compiler_params=pltpu.CompilerParams(needs_layout_passes=False,
                                             use_tc_tiling_on_sc=False),
    )
    def body(*args):
        n_al = 2 if H == 1 else 2 * n_pass
        hps = args[0:n_pass]
        als_all = args[n_pass:n_pass + n_al]
        pk_h = args[n_pass + n_al]
        outs = args[n_pass + n_al + 1:n_pass + n_al + 1 + n_pass]
        refs = args[n_pass + n_al + 1 + n_pass:]
        pk2 = refs[0]
        srcu, dstu = refs[1:4], refs[4:7]
        gbuf, sbuf = refs[7:10], refs[10:13]
        if H == 1:
            aux_s, aux_d = refs[13], refs[14]
            exb = refs[15:18]
            acc = refs[18]
            sem_g, sem_s = refs[19:22], refs[22:25]
        else:
            asb, adb = refs[13:16], refs[16:19]
            acc = refs[19]
            sem_g, sem_s = refs[20:23], refs[23:26]
            sem_as, sem_ad = refs[26:29], refs[29:32]
        cid = lax.axis_index("c")
        sid = lax.axis_index("s")
        wid = sid * 2 + cid
        if nb_split is None:
            pltpu.sync_copy(pk_h.at[wid], pk2)
            nbr = NB
        else:
            nba, nbb = nb_split
            blk0 = (wid // 2) * (nba + nbb) + (wid % 2) * nba
            pltpu.sync_copy(pk_h.at[pl.ds(blk0, nb_max)], pk2)
            nbr = jnp.where(cid == 0, nba, nbb)
        if H == 1:
            pltpu.sync_copy(als_all[0], aux_s)
            pltpu.sync_copy(als_all[1], aux_d)
        row0 = sid * ROWS_PER_TILE
        lanes = lax.iota(jnp.int32, 16)
        z16 = jnp.zeros((16,), jnp.float32)

        def zero_acc():
            for r in range(B):
                for v in range(AW // 16):
                    sbuf[0][r, pl.ds(v * 16, 16)] = z16
            for k in range(ROWS_PER_TILE // B):
                pltpu.sync_copy(sbuf[0], acc.at[pl.ds(row0 + k * B, B)])
            plsc.subcore_barrier()

        def run_pass(hp, als, ald, out):
            def unpack(i_blk, r):
                @plsc.parallel_loop(0, B, step=16, unroll=4)
                def _(o):
                    p = pk2[i_blk, pl.ds(o, 16)]
                    si = lax.bitwise_and(p, PACK - 1)
                    di = lax.shift_right_logical(p, 14)
                    srcu[r][0, pl.ds(o, 16)] = si
                    dstu[r][0, pl.ds(o, 16)] = di
                    if H == 1:
                        a = (plsc.load_gather(aux_s, [si])
                             + plsc.load_gather(aux_d, [di]))
                        ex = jnp.exp(jnp.where(a >= 0, a, a * jnp.float32(0.2)))
                        exb[r][pl.ds(o, 16)] = ex

            def fire_gather(r):
                pltpu.async_copy(hp.at[srcu[r].at[0]], gbuf[r], sem_g[r])
                if H > 1:
                    pltpu.async_copy(als.at[srcu[r].at[0]], asb[r], sem_as[r])
                    pltpu.async_copy(ald.at[dstu[r].at[0]], adb[r], sem_ad[r])

            def wait_gather(r):
                pltpu.make_async_copy(hp.at[srcu[r].at[0]], gbuf[r],
                                      sem_g[r]).wait()
                if H > 1:
                    pltpu.make_async_copy(als.at[srcu[r].at[0]], asb[r],
                                          sem_as[r]).wait()
                    pltpu.make_async_copy(ald.at[dstu[r].at[0]], adb[r],
                                          sem_ad[r]).wait()

            def compute(r):
                if H > 1:
                    @plsc.parallel_loop(0, B, unroll=8)
                    def _(b):
                        arow = asb[r][b, pl.ds(0, 16)] + adb[r][b, pl.ds(0, 16)]
                        ex = jnp.exp(jnp.where(arow >= 0, arow,
                                               arow * jnp.float32(0.2)))
                        sbuf[r][b, pl.ds(WP, 16)] = ex

                @plsc.parallel_loop(0, B, unroll=8)
                def _(b):
                    if H > 1:
                        exdup = sbuf[r][b, pl.ds(WP, 16)]
                    else:
                        exdup = plsc.load_gather(
                            exb[r], [jnp.full((16,), b, jnp.int32)])
                        if AW > WP:
                            sbuf[r][b, pl.ds(WP, 16)] = jnp.where(
                                lanes == 0, exdup, jnp.float32(0.0))
                    for v in range(WP // 16):
                        sbuf[r][b, pl.ds(v * 16, 16)] = (
                            gbuf[r][b, pl.ds(v * 16, 16)] * exdup)

            zero_acc()
            unpack(jnp.int32(0), 0)
            fire_gather(0)

            def step(j, carry):
                for p in range(3):
                    i_blk = j * 3 + p
                    r1 = (p + 1) % 3
                    unpack(jnp.minimum(i_blk + 1, nbr - 1), r1)
                    fire_gather(r1)
                    wait_gather(p)
                    compute(p)
                    pltpu.sync_copy(sbuf[p], acc.at[dstu[p].at[0]], add=True)
                return carry

            lax.fori_loop(0, nbr // 3, step, 0)
            wait_gather(0)
            plsc.subcore_barrier()
            for k in range(ROWS_PER_TILE // B):
                r0 = row0 + k * B
                pltpu.sync_copy(acc.at[pl.ds(r0, B)], sbuf[0])
                pltpu.sync_copy(sbuf[0], out.at[cid, pl.ds(r0, B)])

        for ip in range(n_pass):
            if H == 1:
                run_pass(hps[ip], None, None, outs[ip])
            else:
                run_pass(hps[ip], als_all[2 * ip], als_all[2 * ip + 1],
                         outs[ip])

    return body


_sc_l1 = _make_sc_gat(H=4, WP=80, AW=96, B=64, NB=162, dual=True, nb_split=(144, 192))
_sc_small = _make_sc_gat(H=1, WP=16, AW=16, B=96, NB=108, dual=False)
_sc_l5 = _make_sc_gat(H=1, WP=64, AW=80, B=64, NB=162, dual=True, nb_split=(144, 192))

_R = 512


def _tc_matmul_l1(xp, Wc):
    K, M = Wc.shape

    def tc_body(x_ref, w_ref, oha, ohb, oasa, oada, oasb, oadb):
        t = jnp.dot(x_ref[...], w_ref[...], preferred_element_type=jnp.float32)
        oha[...] = t[:, 0:80]
        ohb[...] = t[:, 80:160]
        oasa[...] = jnp.concatenate([t[:, 160:164]] * 4, axis=1)
        oasb[...] = jnp.concatenate([t[:, 164:168]] * 4, axis=1)
        oada[...] = jnp.concatenate([t[:, 176:180]] * 4, axis=1)
        oadb[...] = jnp.concatenate([t[:, 180:184]] * 4, axis=1)

    s16 = jax.ShapeDtypeStruct((NPAD, 16), jnp.float32)
    s80 = jax.ShapeDtypeStruct((NPAD, 80), jnp.float32)
    bs16 = pl.BlockSpec((_R, 16), lambda i: (i, 0))
    bs80 = pl.BlockSpec((_R, 80), lambda i: (i, 0))
    return pl.pallas_call(
        tc_body,
        grid=(NPAD // _R,),
        in_specs=[pl.BlockSpec((_R, K), lambda i: (i, 0)),
                  pl.BlockSpec((K, M), lambda i: (0, 0))],
        out_specs=[bs80, bs80, bs16, bs16, bs16, bs16],
        out_shape=[s80, s80, s16, s16, s16, s16],
    )(xp, Wc)


def _tc_combine_l1(pa, pb, bp, Wc, ones_col):
    K, M = Wc.shape

    def _half(p_ref):
        acc = p_ref[0] + p_ref[1]
        den = jnp.concatenate([acc[:, 80:84]] * 20, axis=1)
        return acc[:, :80] / (den + 1e-16)

    def tc_body(pa_ref, pb_ref, b_ref, w_ref, o_ref):
        y = jnp.concatenate([_half(pa_ref), _half(pb_ref)], axis=1) + b_ref[...]
        y = jnp.maximum(y, 0.0)
        o = jnp.dot(y, w_ref[...], preferred_element_type=jnp.float32)
        cols = lax.broadcasted_iota(jnp.int32, o.shape, 1)
        o_ref[...] = jnp.where(cols == ones_col, 1.0, o)

    return pl.pallas_call(
        tc_body,
        grid=(NPAD // _R,),
        in_specs=[pl.BlockSpec((2, _R, 96), lambda i: (0, i, 0)),
                  pl.BlockSpec((2, _R, 96), lambda i: (0, i, 0)),
                  pl.BlockSpec((1, 160), lambda i: (0, 0)),
                  pl.BlockSpec((K, M), lambda i: (0, 0))],
        out_specs=pl.BlockSpec((_R, M), lambda i: (i, 0)),
        out_shape=jax.ShapeDtypeStruct((NPAD, M), jnp.float32),
    )(pa, pb, bp, Wc)


def _tc_combine(parts, bp, Wc, AW, den_col, ones_col):
    K, M = Wc.shape

    def tc_body(p_ref, b_ref, w_ref, o_ref):
        acc = p_ref[0] + p_ref[1]
        y = acc[:, :K] / (acc[:, den_col:den_col + 1] + 1e-16) + b_ref[...]
        y = jnp.maximum(y, 0.0)
        o = jnp.dot(y, w_ref[...], preferred_element_type=jnp.float32)
        if ones_col is not None:
            cols = lax.broadcasted_iota(jnp.int32, o.shape, 1)
            o = jnp.where(cols == ones_col, 1.0, o)
        o_ref[...] = o

    return pl.pallas_call(
        tc_body,
        grid=(NPAD // _R,),
        in_specs=[pl.BlockSpec((2, _R, AW), lambda i: (0, i, 0)),
                  pl.BlockSpec((1, K), lambda i: (0, 0)),
                  pl.BlockSpec((K, M), lambda i: (0, 0))],
        out_specs=pl.BlockSpec((_R, M), lambda i: (i, 0)),
        out_shape=jax.ShapeDtypeStruct((NPAD, M), jnp.float32),
    )(parts, bp, Wc)


def _tc_combine_l4(parts, bp, Wc):
    K, M = Wc.shape

    def tc_body(p_ref, b_ref, w_ref, oha, ohb, oals, oald):
        acc = p_ref[0] + p_ref[1]
        y = acc[:, :K] / (acc[:, 3:4] + 1e-16) + b_ref[...]
        y = jnp.maximum(y, 0.0)
        o = jnp.dot(y, w_ref[...], preferred_element_type=jnp.float32)
        oha[...] = o[:, 0:64]
        ohb[...] = o[:, 64:128]
        oals[...] = o[:, 128]
        oald[...] = o[:, 144]

    s64 = jax.ShapeDtypeStruct((NPAD, 64), jnp.float32)
    s1 = jax.ShapeDtypeStruct((NPAD,), jnp.float32)
    bs64 = pl.BlockSpec((_R, 64), lambda i: (i, 0))
    bs1 = pl.BlockSpec((_R,), lambda i: (i,))
    return pl.pallas_call(
        tc_body,
        grid=(NPAD // _R,),
        in_specs=[pl.BlockSpec((2, _R, 16), lambda i: (0, i, 0)),
                  pl.BlockSpec((1, K), lambda i: (0, 0)),
                  pl.BlockSpec((K, M), lambda i: (0, 0))],
        out_specs=[bs64, bs64, bs1, bs1],
        out_shape=[s64, s64, s1, s1],
    )(parts, bp, Wc)


def _tc_final_l5(pa, pb, bp):
    def tc_body(pa_ref, pb_ref, b_ref, o_ref):
        na = pa_ref[0] + pa_ref[1]
        nb = pb_ref[0] + pb_ref[1]
        den = na[:, 64:65] + 1e-16
        y = jnp.concatenate([na[:, :64], nb[:, :64]], axis=1) / den
        o_ref[...] = y + b_ref[...]

    return pl.pallas_call(
        tc_body,
        grid=(NPAD // _R,),
        in_specs=[pl.BlockSpec((2, _R, 80), lambda i: (0, i, 0)),
                  pl.BlockSpec((2, _R, 80), lambda i: (0, i, 0)),
                  pl.BlockSpec((1, 128), lambda i: (0, 0))],
        out_specs=pl.BlockSpec((_R, 128), lambda i: (i, 0)),
        out_shape=jax.ShapeDtypeStruct((NPAD, 128), jnp.float32),
    )(pa, pb, bp)


def _pad_w(W, rows, cols):
    return jnp.zeros((rows, cols), jnp.float32).at[:W.shape[0], :W.shape[1]].set(W)


def kernel(x, edge_index, W1, as1, ad1, b1, W2, as2, ad2, b2, W3, as3, ad3, b3,
           W4, as4, ad4, b4, W5, as5, ad5, b5):
    sl = jnp.arange(N, dtype=edge_index.dtype)
    src = jnp.concatenate([edge_index[0], sl])
    dst = jnp.concatenate([edge_index[1], sl])
    pkv = dst * PACK + src
    pad = N * PACK + N
    pka = jnp.full((16 * 336 * 64,), pad, jnp.int32).at[:E0].set(pkv)
    pk3a = pka.reshape(16 * 336, 64)
    pkb = jnp.full((EPAD,), pad, jnp.int32).at[:E0].set(pkv)
    pk3b = pkb.reshape(NW, 108, 96)

    W1r = W1.reshape(128, 8, 20)
    W1pa = W1r[:, 0:4].transpose(0, 2, 1).reshape(128, 80)
    W1pb = W1r[:, 4:8].transpose(0, 2, 1).reshape(128, 80)
    Ms1 = (W1r * as1[None]).sum(-1)
    Md1 = (W1r * ad1[None]).sum(-1)
    z8 = jnp.zeros((128, 8), jnp.float32)
    Wcat0 = jnp.concatenate([W1pa, W1pb, Ms1, z8, Md1, z8], axis=1)

    rowidx = jnp.array(
        [(j % 4) * 20 + j // 4 for j in range(80)]
        + [(4 + j % 4) * 20 + j // 4 for j in range(80)], jnp.int32)
    W2cat = jnp.take(W2, rowidx, axis=0)
    b1cat = jnp.take(b1, rowidx)[None]
    Wcat1 = jnp.concatenate([
        _pad_w(W2cat, 160, 16),
        _pad_w((W2cat @ as2[0])[:, None], 160, 16),
        _pad_w((W2cat @ ad2[0])[:, None], 160, 16)], axis=1)
    b2p = _pad_w(b2[None], 1, 16)

    Wcat2 = jnp.concatenate([
        _pad_w(W3, 16, 16),
        _pad_w((W3 @ as3[0])[:, None], 16, 16),
        _pad_w((W3 @ ad3[0])[:, None], 16, 16)], axis=1)
    b3p = _pad_w(b3[None], 1, 16)

    Wcat3 = jnp.concatenate([
        _pad_w(W4, 16, 16),
        _pad_w((W4 @ as4[0])[:, None], 16, 16),
        _pad_w((W4 @ ad4[0])[:, None], 16, 16)], axis=1)
    b4p = _pad_w(b4[None], 1, 16)

    Wcat4 = jnp.concatenate([
        _pad_w(W5, 16, 128),
        _pad_w((W5 @ as5[0])[:, None], 16, 16),
        _pad_w((W5 @ ad5[0])[:, None], 16, 16)], axis=1)
    b5p = b5[None]

    xp = jnp.zeros((NPAD, 128), jnp.float32).at[:N].set(x)

    h1a, h1b, asa, ada, asb, adb = _tc_matmul_l1(xp, Wcat0)
    p1a, p1b = _sc_l1(h1a, h1b, asa, ada, asb, adb, pk3a)
    t2 = _tc_combine_l1(p1a, p1b, b1cat, Wcat1, ones_col=10)
    p2 = _sc_small(t2[:, :16], t2[:, 16], t2[:, 32], pk3b)
    t3 = _tc_combine(p2, b2p, Wcat2, AW=16, den_col=10, ones_col=5)
    p3 = _sc_small(t3[:, :16], t3[:, 16], t3[:, 32], pk3b)
    t4 = _tc_combine(p3, b3p, Wcat3, AW=16, den_col=5, ones_col=3)
    p4 = _sc_small(t4[:, :16], t4[:, 16], t4[:, 32], pk3b)
    h5a, h5b, als5, ald5 = _tc_combine_l4(p4, b4p, Wcat4)
    p5a, p5b = _sc_l5(h5a, h5b, als5, ald5, pk3a)
    y5 = _tc_final_l5(p5a, p5b, b5p)
    return y5[:N]


# speedup vs baseline: 1.8445x; 1.8445x over previous
import functools

import jax
import jax.numpy as jnp
from jax import lax
from jax.experimental import pallas as pl
from jax.experimental.pallas import tpu as pltpu
from jax.experimental.pallas import tpu_sc as plsc

N = 10000
NPAD = 10240
E0 = 330000
NW = 32
EPT = 10368
EPAD = NW * EPT
ROWS_PER_TILE = NPAD // 16
PACK = 16384


def _make_sc_gat(H, WP, AW, B, NB, dual):
    assert NB % 3 == 0
    n_pass = 2 if dual else 1
    mesh = plsc.VectorSubcoreMesh(core_axis_name="c", subcore_axis_name="s")
    scratch = [pltpu.VMEM((NB, B), jnp.int32)]
    scratch += [pltpu.VMEM((1, B), jnp.int32)] * 6
    scratch += [pltpu.VMEM((B, WP), jnp.float32)] * 3
    scratch += [pltpu.VMEM((B, AW), jnp.float32)] * 3
    if H == 1:
        scratch += [pltpu.VMEM((NPAD,), jnp.float32)] * 2
        scratch += [pltpu.VMEM((B * 16,), jnp.float32)] * 3
    else:
        scratch += [pltpu.VMEM((B, 16), jnp.float32)] * 6
    scratch += [pltpu.VMEM_SHARED((NPAD, AW), jnp.float32)]
    scratch += [pltpu.SemaphoreType.DMA] * (6 + (6 if H > 1 else 0))

    out_one = jax.ShapeDtypeStruct((2, NPAD, AW), jnp.float32)

    @functools.partial(
        pl.kernel,
        out_type=tuple([out_one] * n_pass) if dual else out_one,
        mesh=mesh,
        scratch_types=scratch,
        compiler_params=pltpu.CompilerParams(needs_layout_passes=False,
                                             use_tc_tiling_on_sc=False),
    )
    def body(*args):
        n_al = 2 if H == 1 else 2 * n_pass
        hps = args[0:n_pass]
        als_all = args[n_pass:n_pass + n_al]
        pk_h = args[n_pass + n_al]
        outs = args[n_pass + n_al + 1:n_pass + n_al + 1 + n_pass]
        refs = args[n_pass + n_al + 1 + n_pass:]
        pk2 = refs[0]
        srcu, dstu = refs[1:4], refs[4:7]
        gbuf, sbuf = refs[7:10], refs[10:13]
        if H == 1:
            aux_s, aux_d = refs[13], refs[14]
            exb = refs[15:18]
            acc = refs[18]
            sem_g, sem_s = refs[19:22], refs[22:25]
        else:
            asb, adb = refs[13:16], refs[16:19]
            acc = refs[19]
            sem_g, sem_s = refs[20:23], refs[23:26]
            sem_as, sem_ad = refs[26:29], refs[29:32]
        cid = lax.axis_index("c")
        sid = lax.axis_index("s")
        wid = sid * 2 + cid
        pltpu.sync_copy(pk_h.at[wid], pk2)
        if H == 1:
            pltpu.sync_copy(als_all[0], aux_s)
            pltpu.sync_copy(als_all[1], aux_d)
        row0 = sid * ROWS_PER_TILE
        lanes = lax.iota(jnp.int32, 16)
        z16 = jnp.zeros((16,), jnp.float32)

        def zero_acc():
            for r in range(B):
                for v in range(AW // 16):
                    sbuf[0][r, pl.ds(v * 16, 16)] = z16
            for k in range(ROWS_PER_TILE // B):
                pltpu.sync_copy(sbuf[0], acc.at[pl.ds(row0 + k * B, B)])
            plsc.subcore_barrier()

        def run_pass(hp, als, ald, out):
            def unpack(i_blk, r):
                @plsc.parallel_loop(0, B, step=16, unroll=4)
                def _(o):
                    p = pk2[i_blk, pl.ds(o, 16)]
                    si = lax.bitwise_and(p, PACK - 1)
                    di = lax.shift_right_logical(p, 14)
                    srcu[r][0, pl.ds(o, 16)] = si
                    dstu[r][0, pl.ds(o, 16)] = di
                    if H == 1:
                        a = (plsc.load_gather(aux_s, [si])
                             + plsc.load_gather(aux_d, [di]))
                        ex = jnp.exp(jnp.where(a >= 0, a, a * jnp.float32(0.2)))
                        exb[r][pl.ds(o, 16)] = ex

            def fire_gather(r):
                pltpu.async_copy(hp.at[srcu[r].at[0]], gbuf[r], sem_g[r])
                if H > 1:
                    pltpu.async_copy(als.at[srcu[r].at[0]], asb[r], sem_as[r])
                    pltpu.async_copy(ald.at[dstu[r].at[0]], adb[r], sem_ad[r])

            def wait_gather(r):
                pltpu.make_async_copy(hp.at[srcu[r].at[0]], gbuf[r],
                                      sem_g[r]).wait()
                if H > 1:
                    pltpu.make_async_copy(als.at[srcu[r].at[0]], asb[r],
                                          sem_as[r]).wait()
                    pltpu.make_async_copy(ald.at[dstu[r].at[0]], adb[r],
                                          sem_ad[r]).wait()

            def compute(r):
                if H > 1:
                    @plsc.parallel_loop(0, B, unroll=8)
                    def _(b):
                        arow = asb[r][b, pl.ds(0, 16)] + adb[r][b, pl.ds(0, 16)]
                        ex = jnp.exp(jnp.where(arow >= 0, arow,
                                               arow * jnp.float32(0.2)))
                        sbuf[r][b, pl.ds(WP, 16)] = ex

                @plsc.parallel_loop(0, B, unroll=8)
                def _(b):
                    if H > 1:
                        exdup = sbuf[r][b, pl.ds(WP, 16)]
                    else:
                        exdup = plsc.load_gather(
                            exb[r], [jnp.full((16,), b, jnp.int32)])
                        if AW > WP:
                            sbuf[r][b, pl.ds(WP, 16)] = jnp.where(
                                lanes == 0, exdup, jnp.float32(0.0))
                    for v in range(WP // 16):
                        sbuf[r][b, pl.ds(v * 16, 16)] = (
                            gbuf[r][b, pl.ds(v * 16, 16)] * exdup)

            zero_acc()
            unpack(jnp.int32(0), 0)
            fire_gather(0)

            def step(j, carry):
                for p in range(3):
                    i_blk = j * 3 + p
                    r1 = (p + 1) % 3
                    unpack(jnp.minimum(i_blk + 1, NB - 1), r1)
                    fire_gather(r1)
                    wait_gather(p)
                    compute(p)
                    pltpu.sync_copy(sbuf[p], acc.at[dstu[p].at[0]], add=True)
                return carry

            lax.fori_loop(0, NB // 3, step, 0)
            wait_gather(0)
            plsc.subcore_barrier()
            for k in range(ROWS_PER_TILE // B):
                r0 = row0 + k * B
                pltpu.sync_copy(acc.at[pl.ds(r0, B)], sbuf[0])
                pltpu.sync_copy(sbuf[0], out.at[cid, pl.ds(r0, B)])

        for ip in range(n_pass):
            if H == 1:
                run_pass(hps[ip], None, None, outs[ip])
            else:
                run_pass(hps[ip], als_all[2 * ip], als_all[2 * ip + 1],
                         outs[ip])

    return body


_sc_l1 = _make_sc_gat(H=4, WP=80, AW=96, B=64, NB=162, dual=True)
_sc_small = _make_sc_gat(H=1, WP=16, AW=16, B=96, NB=108, dual=False)
_sc_l5 = _make_sc_gat(H=1, WP=64, AW=80, B=64, NB=162, dual=True)

_R = 512


def _tc_matmul_l1(xp, Wc):
    K, M = Wc.shape

    def tc_body(x_ref, w_ref, oha, ohb, oasa, oada, oasb, oadb):
        t = jnp.dot(x_ref[...], w_ref[...], preferred_element_type=jnp.float32)
        oha[...] = t[:, 0:80]
        ohb[...] = t[:, 80:160]
        oasa[...] = jnp.concatenate([t[:, 160:164]] * 4, axis=1)
        oasb[...] = jnp.concatenate([t[:, 164:168]] * 4, axis=1)
        oada[...] = jnp.concatenate([t[:, 176:180]] * 4, axis=1)
        oadb[...] = jnp.concatenate([t[:, 180:184]] * 4, axis=1)

    s16 = jax.ShapeDtypeStruct((NPAD, 16), jnp.float32)
    s80 = jax.ShapeDtypeStruct((NPAD, 80), jnp.float32)
    bs16 = pl.BlockSpec((_R, 16), lambda i: (i, 0))
    bs80 = pl.BlockSpec((_R, 80), lambda i: (i, 0))
    return pl.pallas_call(
        tc_body,
        grid=(NPAD // _R,),
        in_specs=[pl.BlockSpec((_R, K), lambda i: (i, 0)),
                  pl.BlockSpec((K, M), lambda i: (0, 0))],
        out_specs=[bs80, bs80, bs16, bs16, bs16, bs16],
        out_shape=[s80, s80, s16, s16, s16, s16],
    )(xp, Wc)


def _tc_combine_l1(pa, pb, bp, Wc, ones_col):
    K, M = Wc.shape

    def _half(p_ref):
        acc = p_ref[0] + p_ref[1]
        den = jnp.concatenate([acc[:, 80:84]] * 20, axis=1)
        return acc[:, :80] / (den + 1e-16)

    def tc_body(pa_ref, pb_ref, b_ref, w_ref, o_ref):
        y = jnp.concatenate([_half(pa_ref), _half(pb_ref)], axis=1) + b_ref[...]
        y = jnp.maximum(y, 0.0)
        o = jnp.dot(y, w_ref[...], preferred_element_type=jnp.float32)
        cols = lax.broadcasted_iota(jnp.int32, o.shape, 1)
        o_ref[...] = jnp.where(cols == ones_col, 1.0, o)

    return pl.pallas_call(
        tc_body,
        grid=(NPAD // _R,),
        in_specs=[pl.BlockSpec((2, _R, 96), lambda i: (0, i, 0)),
                  pl.BlockSpec((2, _R, 96), lambda i: (0, i, 0)),
                  pl.BlockSpec((1, 160), lambda i: (0, 0)),
                  pl.BlockSpec((K, M), lambda i: (0, 0))],
        out_specs=pl.BlockSpec((_R, M), lambda i: (i, 0)),
        out_shape=jax.ShapeDtypeStruct((NPAD, M), jnp.float32),
    )(pa, pb, bp, Wc)


def _tc_combine(parts, bp, Wc, AW, den_col, ones_col):
    K, M = Wc.shape

    def tc_body(p_ref, b_ref, w_ref, o_ref):
        acc = p_ref[0] + p_ref[1]
        y = acc[:, :K] / (acc[:, den_col:den_col + 1] + 1e-16) + b_ref[...]
        y = jnp.maximum(y, 0.0)
        o = jnp.dot(y, w_ref[...], preferred_element_type=jnp.float32)
        if ones_col is not None:
            cols = lax.broadcasted_iota(jnp.int32, o.shape, 1)
            o = jnp.where(cols == ones_col, 1.0, o)
        o_ref[...] = o

    return pl.pallas_call(
        tc_body,
        grid=(NPAD // _R,),
        in_specs=[pl.BlockSpec((2, _R, AW), lambda i: (0, i, 0)),
                  pl.BlockSpec((1, K), lambda i: (0, 0)),
                  pl.BlockSpec((K, M), lambda i: (0, 0))],
        out_specs=pl.BlockSpec((_R, M), lambda i: (i, 0)),
        out_shape=jax.ShapeDtypeStruct((NPAD, M), jnp.float32),
    )(parts, bp, Wc)


def _tc_combine_l4(parts, bp, Wc):
    K, M = Wc.shape

    def tc_body(p_ref, b_ref, w_ref, oha, ohb, oals, oald):
        acc = p_ref[0] + p_ref[1]
        y = acc[:, :K] / (acc[:, 3:4] + 1e-16) + b_ref[...]
        y = jnp.maximum(y, 0.0)
        o = jnp.dot(y, w_ref[...], preferred_element_type=jnp.float32)
        oha[...] = o[:, 0:64]
        ohb[...] = o[:, 64:128]
        oals[...] = o[:, 128]
        oald[...] = o[:, 144]

    s64 = jax.ShapeDtypeStruct((NPAD, 64), jnp.float32)
    s1 = jax.ShapeDtypeStruct((NPAD,), jnp.float32)
    bs64 = pl.BlockSpec((_R, 64), lambda i: (i, 0))
    bs1 = pl.BlockSpec((_R,), lambda i: (i,))
    return pl.pallas_call(
        tc_body,
        grid=(NPAD // _R,),
        in_specs=[pl.BlockSpec((2, _R, 16), lambda i: (0, i, 0)),
                  pl.BlockSpec((1, K), lambda i: (0, 0)),
                  pl.BlockSpec((K, M), lambda i: (0, 0))],
        out_specs=[bs64, bs64, bs1, bs1],
        out_shape=[s64, s64, s1, s1],
    )(parts, bp, Wc)


def _tc_final_l5(pa, pb, bp):
    def tc_body(pa_ref, pb_ref, b_ref, o_ref):
        na = pa_ref[0] + pa_ref[1]
        nb = pb_ref[0] + pb_ref[1]
        den = na[:, 64:65] + 1e-16
        y = jnp.concatenate([na[:, :64], nb[:, :64]], axis=1) / den
        o_ref[...] = y + b_ref[...]

    return pl.pallas_call(
        tc_body,
        grid=(NPAD // _R,),
        in_specs=[pl.BlockSpec((2, _R, 80), lambda i: (0, i, 0)),
                  pl.BlockSpec((2, _R, 80), lambda i: (0, i, 0)),
                  pl.BlockSpec((1, 128), lambda i: (0, 0))],
        out_specs=pl.BlockSpec((_R, 128), lambda i: (i, 0)),
        out_shape=jax.ShapeDtypeStruct((NPAD, 128), jnp.float32),
    )(pa, pb, bp)


def _pad_w(W, rows, cols):
    return jnp.zeros((rows, cols), jnp.float32).at[:W.shape[0], :W.shape[1]].set(W)


def kernel(x, edge_index, W1, as1, ad1, b1, W2, as2, ad2, b2, W3, as3, ad3, b3,
           W4, as4, ad4, b4, W5, as5, ad5, b5):
    sl = jnp.arange(N, dtype=edge_index.dtype)
    src = jnp.concatenate([edge_index[0], sl])
    dst = jnp.concatenate([edge_index[1], sl])
    pk = jnp.full((EPAD,), N * PACK + N, jnp.int32)
    pk = pk.at[:E0].set(dst * PACK + src)
    pk3a = pk.reshape(NW, 162, 64)
    pk3b = pk.reshape(NW, 108, 96)

    W1r = W1.reshape(128, 8, 20)
    W1pa = W1r[:, 0:4].transpose(0, 2, 1).reshape(128, 80)
    W1pb = W1r[:, 4:8].transpose(0, 2, 1).reshape(128, 80)
    Ms1 = (W1r * as1[None]).sum(-1)
    Md1 = (W1r * ad1[None]).sum(-1)
    z8 = jnp.zeros((128, 8), jnp.float32)
    Wcat0 = jnp.concatenate([W1pa, W1pb, Ms1, z8, Md1, z8], axis=1)

    rowidx = jnp.array(
        [(j % 4) * 20 + j // 4 for j in range(80)]
        + [(4 + j % 4) * 20 + j // 4 for j in range(80)], jnp.int32)
    W2cat = jnp.take(W2, rowidx, axis=0)
    b1cat = jnp.take(b1, rowidx)[None]
    Wcat1 = jnp.concatenate([
        _pad_w(W2cat, 160, 16),
        _pad_w((W2cat @ as2[0])[:, None], 160, 16),
        _pad_w((W2cat @ ad2[0])[:, None], 160, 16)], axis=1)
    b2p = _pad_w(b2[None], 1, 16)

    Wcat2 = jnp.concatenate([
        _pad_w(W3, 16, 16),
        _pad_w((W3 @ as3[0])[:, None], 16, 16),
        _pad_w((W3 @ ad3[0])[:, None], 16, 16)], axis=1)
    b3p = _pad_w(b3[None], 1, 16)

    Wcat3 = jnp.concatenate([
        _pad_w(W4, 16, 16),
        _pad_w((W4 @ as4[0])[:, None], 16, 16),
        _pad_w((W4 @ ad4[0])[:, None], 16, 16)], axis=1)
    b4p = _pad_w(b4[None], 1, 16)

    Wcat4 = jnp.concatenate([
        _pad_w(W5, 16, 128),
        _pad_w((W5 @ as5[0])[:, None], 16, 16),
        _pad_w((W5 @ ad5[0])[:, None], 16, 16)], axis=1)
    b5p = b5[None]

    xp = jnp.zeros((NPAD, 128), jnp.float32).at[:N].set(x)

    h1a, h1b, asa, ada, asb, adb = _tc_matmul_l1(xp, Wcat0)
    p1a, p1b = _sc_l1(h1a, h1b, asa, ada, asb, adb, pk3a)
    t2 = _tc_combine_l1(p1a, p1b, b1cat, Wcat1, ones_col=10)
    p2 = _sc_small(t2[:, :16], t2[:, 16], t2[:, 32], pk3b)
    t3 = _tc_combine(p2, b2p, Wcat2, AW=16, den_col=10, ones_col=5)
    p3 = _sc_small(t3[:, :16], t3[:, 16], t3[:, 32], pk3b)
    t4 = _tc_combine(p3, b3p, Wcat3, AW=16, den_col=5, ones_col=3)
    p4 = _sc_small(t4[:, :16], t4[:, 16], t4[:, 32], pk3b)
    h5a, h5b, als5, ald5 = _tc_combine_l4(p4, b4p, Wcat4)
    p5a, p5b = _sc_l5(h5a, h5b, als5, ald5, pk3a)
    y5 = _tc_final_l5(p5a, p5b, b5p)
    return y5[:N]
